# baseline (speedup 1.0000x reference)
import functools

import jax
import jax.numpy as jnp
from jax import lax
from jax.experimental import pallas as pl
from jax.experimental.pallas import tpu as pltpu
from jax.experimental.pallas import tpu_sc as plsc

_NE = 1024
_ED = 128
_NBIT = 10
_BETA = 0.25
_SNR_DB = 10.0

_TOK_BLK = 4096

_NC, _NS = 2, 16
_NW = _NC * _NS
_GCHUNK = 160


def _vq_argmin_body(zf_ref, embt_ref, zsq_ref, am_ref, om_ref, ridx_ref,
                    loss_ref):
    zf = zf_ref[...]
    embt = embt_ref[...]
    mm = jnp.dot(zf, embt, preferred_element_type=jnp.float32)
    zsq = jnp.transpose(zsq_ref[0], (1, 0))
    ssq = jnp.sum(embt * embt, axis=0, keepdims=True)
    d = (zsq + ssq) - 2.0 * mm
    dmin = jnp.min(d, axis=1, keepdims=True)
    ids = lax.broadcasted_iota(jnp.int32, (1, _NE), 1)
    idx = jnp.min(jnp.where(d == dmin, ids, _NE), axis=1, keepdims=True)
    idx_l = jnp.transpose(idx, (1, 0))
    ridx_ref[...] = ((idx_l & am_ref[0]) | om_ref[0])[None]

    @pl.when(pl.program_id(0) == 0)
    def _init():
        loss_ref[...] = jnp.zeros_like(loss_ref)

    loss_ref[...] += jnp.sum(dmin, axis=0, keepdims=True)


def _tc_vq(zp, embt, zsq3, am, om, blk0, nblk_i):
    return pl.pallas_call(
        _vq_argmin_body,
        grid=(nblk_i,),
        in_specs=[
            pl.BlockSpec((_TOK_BLK, _ED), lambda i: (blk0 + i, 0)),
            pl.BlockSpec((_ED, _NE), lambda i: (0, 0)),
            pl.BlockSpec((1, 1, _TOK_BLK), lambda i: (blk0 + i, 0, 0)),
            pl.BlockSpec((1, 1, _TOK_BLK), lambda i: (blk0 + i, 0, 0)),
            pl.BlockSpec((1, 1, _TOK_BLK), lambda i: (blk0 + i, 0, 0)),
        ],
        out_specs=[
            pl.BlockSpec((1, 1, _TOK_BLK), lambda i: (i, 0, 0)),
            pl.BlockSpec((1, 1), lambda i: (0, 0)),
        ],
        out_shape=[
            jax.ShapeDtypeStruct((nblk_i, 1, _TOK_BLK), jnp.int32),
            jax.ShapeDtypeStruct((1, 1), jnp.float32),
        ],
    )(zp, embt, zsq3, am, om)


def _sc_gather(ridx, emb):
    ntok = ridx.shape[0]
    bpw = ntok // _NW
    nch = bpw // _GCHUNK
    mesh = plsc.VectorSubcoreMesh(core_axis_name="c", subcore_axis_name="s")

    @functools.partial(
        pl.kernel,
        mesh=mesh,
        out_type=jax.ShapeDtypeStruct((ntok, _ED), jnp.float32),
        scratch_types=[
            pltpu.VMEM((bpw,), jnp.int32),
            pltpu.VMEM((_GCHUNK, _ED), jnp.float32),
            pltpu.VMEM((_GCHUNK, _ED), jnp.float32),
            pltpu.SemaphoreType.DMA,
            pltpu.SemaphoreType.DMA,
        ],
    )
    def gather_k(ridx_hbm, emb_hbm, out_hbm, idx_v, buf0, buf1, sem0, sem1):
        wid = lax.axis_index("s") * _NC + lax.axis_index("c")
        base = wid * bpw
        pltpu.sync_copy(ridx_hbm.at[pl.ds(base, bpw)], idx_v)

        def chunk2(i, carry):
            o0 = (2 * i) * _GCHUNK
            o1 = o0 + _GCHUNK
            a0 = pltpu.async_copy(emb_hbm.at[idx_v.at[pl.ds(o0, _GCHUNK)]],
                                  buf0, sem0)
            a1 = pltpu.async_copy(emb_hbm.at[idx_v.at[pl.ds(o1, _GCHUNK)]],
                                  buf1, sem1)
            a0.wait()
            pltpu.sync_copy(buf0, out_hbm.at[pl.ds(base + o0, _GCHUNK)])
            a1.wait()
            pltpu.sync_copy(buf1, out_hbm.at[pl.ds(base + o1, _GCHUNK)])
            return carry

        lax.fori_loop(0, nch // 2, chunk2, 0)

    return gather_k(ridx, emb)


@functools.lru_cache(maxsize=None)
def _channel_masks(nbatch, npos):
    ntok = nbatch * npos
    cpu = jax.devices("cpu")[0]
    with jax.ensure_compile_time_eval(), jax.default_device(cpu):
        shifts = jnp.arange(_NBIT - 1, -1, -1, dtype=jnp.int32)
        snr_linear = 10.0 ** (_SNR_DB / 10.0)
        noise_std = jnp.sqrt(jnp.asarray(0.5 / snr_linear, dtype=jnp.float32))
        noise = jax.random.normal(jax.random.key(1234), (ntok * _NBIT,),
                                  dtype=jnp.float32) * noise_std
        n = noise.reshape(-1, _NBIT)
        pw = jnp.left_shift(jnp.int32(1), shifts)
        keep1 = (1.0 + n) >= 0.0
        make1 = (-1.0 + n) >= 0.0
        and_mask = jnp.sum(jnp.where(keep1, pw, 0), axis=1).astype(jnp.int32)
        or_mask = jnp.sum(jnp.where(make1, pw, 0), axis=1).astype(jnp.int32)
        import numpy as _np
        nblk = ntok // _TOK_BLK
        am = _np.asarray(and_mask).reshape(nbatch, npos).T.reshape(nblk, 1, _TOK_BLK)
        om = _np.asarray(or_mask).reshape(nbatch, npos).T.reshape(nblk, 1, _TOK_BLK)
        return _np.ascontiguousarray(am), _np.ascontiguousarray(om)


def kernel(z, emb):
    b, c, h, w = z.shape
    hw = h * w
    ntok = b * hw
    zp = jnp.transpose(z.reshape(b, c, hw), (2, 0, 1)).reshape(ntok, _ED)
    embt = jnp.transpose(emb)
    am, om = _channel_masks(b, hw)
    am = jnp.asarray(am)
    om = jnp.asarray(om)
    p_split = hw * 2 // 5
    bounds = [0, p_split * b, ntok]
    zsq3 = jnp.sum(zp ** 2, axis=1).reshape(ntok // _TOK_BLK, 1, _TOK_BLK)
    zq_parts, loss_parts = [], []
    for t0, t1 in zip(bounds[:-1], bounds[1:]):
        nblk_i = (t1 - t0) // _TOK_BLK
        ridx_i, ls_i = _tc_vq(zp, embt, zsq3, am, om, t0 // _TOK_BLK, nblk_i)
        zq_parts.append(_sc_gather(ridx_i.reshape(t1 - t0), emb)
                        .reshape((t1 - t0) // b, b, _ED))
        loss_parts.append(ls_i[0, 0])
    loss = (loss_parts[0] + loss_parts[1]) * jnp.float32(
        (1.0 + _BETA) / float(z.size))
    zq = jnp.concatenate(zq_parts, axis=0)
    out = jnp.transpose(zq, (1, 2, 0)).reshape(b, c, h, w)
    return loss, out


# speedup vs baseline: 1.0255x; 1.0255x over previous
import functools

import jax
import jax.numpy as jnp
from jax import lax
from jax.experimental import pallas as pl
from jax.experimental.pallas import tpu as pltpu
from jax.experimental.pallas import tpu_sc as plsc

_NE = 1024
_ED = 128
_NBIT = 10
_BETA = 0.25
_SNR_DB = 10.0

_TOK_BLK = 4096

_NC, _NS = 2, 16
_NW = _NC * _NS
_GCHUNK = 160


def _vq_argmin_body(zf_ref, embt_ref, zsq_ref, am_ref, om_ref, ridx_ref,
                    loss_ref):
    zf = zf_ref[...]
    embt = embt_ref[...]
    mm = jnp.dot(zf, embt, preferred_element_type=jnp.float32)
    zsq = jnp.transpose(zsq_ref[0], (1, 0))
    ssq = jnp.sum(embt * embt, axis=0, keepdims=True)
    d = (zsq + ssq) - 2.0 * mm
    dmin = jnp.min(d, axis=1, keepdims=True)
    ids = lax.broadcasted_iota(jnp.int32, (1, _NE), 1)
    idx = jnp.min(jnp.where(d == dmin, ids, _NE), axis=1, keepdims=True)
    idx_l = jnp.transpose(idx, (1, 0))
    ridx_ref[...] = ((idx_l & am_ref[0]) | om_ref[0])[None]

    @pl.when(pl.program_id(0) == 0)
    def _init():
        loss_ref[...] = jnp.zeros_like(loss_ref)

    loss_ref[...] += jnp.sum(dmin, axis=0, keepdims=True)


def _tc_vq(zp, embt, zsq3, am, om, blk0, nblk_i):
    return pl.pallas_call(
        _vq_argmin_body,
        grid=(nblk_i,),
        in_specs=[
            pl.BlockSpec((_TOK_BLK, _ED), lambda i: (blk0 + i, 0)),
            pl.BlockSpec((_ED, _NE), lambda i: (0, 0)),
            pl.BlockSpec((1, 1, _TOK_BLK), lambda i: (blk0 + i, 0, 0)),
            pl.BlockSpec((1, 1, _TOK_BLK), lambda i: (blk0 + i, 0, 0)),
            pl.BlockSpec((1, 1, _TOK_BLK), lambda i: (blk0 + i, 0, 0)),
        ],
        out_specs=[
            pl.BlockSpec((1, 1, _TOK_BLK), lambda i: (i, 0, 0)),
            pl.BlockSpec((1, 1), lambda i: (0, 0)),
        ],
        out_shape=[
            jax.ShapeDtypeStruct((nblk_i, 1, _TOK_BLK), jnp.int32),
            jax.ShapeDtypeStruct((1, 1), jnp.float32),
        ],
    )(zp, embt, zsq3, am, om)


def _sc_gather(ridx, emb):
    ntok = ridx.shape[0]
    bpw = ntok // _NW
    nch = bpw // _GCHUNK
    mesh = plsc.VectorSubcoreMesh(core_axis_name="c", subcore_axis_name="s")

    @functools.partial(
        pl.kernel,
        mesh=mesh,
        out_type=jax.ShapeDtypeStruct((ntok, _ED), jnp.float32),
        scratch_types=[
            pltpu.VMEM((bpw,), jnp.int32),
            pltpu.VMEM((_GCHUNK, _ED), jnp.float32),
            pltpu.VMEM((_GCHUNK, _ED), jnp.float32),
            pltpu.SemaphoreType.DMA,
            pltpu.SemaphoreType.DMA,
        ],
    )
    def gather_k(ridx_hbm, emb_hbm, out_hbm, idx_v, buf0, buf1, sem0, sem1):
        wid = lax.axis_index("s") * _NC + lax.axis_index("c")
        base = wid * bpw
        pltpu.sync_copy(ridx_hbm.at[pl.ds(base, bpw)], idx_v)

        def chunk2(i, carry):
            o0 = (2 * i) * _GCHUNK
            o1 = o0 + _GCHUNK
            a0 = pltpu.async_copy(emb_hbm.at[idx_v.at[pl.ds(o0, _GCHUNK)]],
                                  buf0, sem0)
            a1 = pltpu.async_copy(emb_hbm.at[idx_v.at[pl.ds(o1, _GCHUNK)]],
                                  buf1, sem1)
            a0.wait()
            pltpu.sync_copy(buf0, out_hbm.at[pl.ds(base + o0, _GCHUNK)])
            a1.wait()
            pltpu.sync_copy(buf1, out_hbm.at[pl.ds(base + o1, _GCHUNK)])
            return carry

        lax.fori_loop(0, nch // 2, chunk2, 0)

    return gather_k(ridx, emb)


@functools.lru_cache(maxsize=None)
def _channel_masks(nbatch, npos):
    ntok = nbatch * npos
    cpu = jax.devices("cpu")[0]
    with jax.ensure_compile_time_eval(), jax.default_device(cpu):
        shifts = jnp.arange(_NBIT - 1, -1, -1, dtype=jnp.int32)
        snr_linear = 10.0 ** (_SNR_DB / 10.0)
        noise_std = jnp.sqrt(jnp.asarray(0.5 / snr_linear, dtype=jnp.float32))
        noise = jax.random.normal(jax.random.key(1234), (ntok * _NBIT,),
                                  dtype=jnp.float32) * noise_std
        n = noise.reshape(-1, _NBIT)
        pw = jnp.left_shift(jnp.int32(1), shifts)
        keep1 = (1.0 + n) >= 0.0
        make1 = (-1.0 + n) >= 0.0
        and_mask = jnp.sum(jnp.where(keep1, pw, 0), axis=1).astype(jnp.int32)
        or_mask = jnp.sum(jnp.where(make1, pw, 0), axis=1).astype(jnp.int32)
        import numpy as _np
        nblk = ntok // _TOK_BLK
        am = _np.asarray(and_mask).reshape(nbatch, npos).T.reshape(nblk, 1, _TOK_BLK)
        om = _np.asarray(or_mask).reshape(nbatch, npos).T.reshape(nblk, 1, _TOK_BLK)
        return _np.ascontiguousarray(am), _np.ascontiguousarray(om)


def kernel(z, emb):
    b, c, h, w = z.shape
    hw = h * w
    ntok = b * hw
    zp = jnp.transpose(z.reshape(b, c, hw), (2, 0, 1)).reshape(ntok, _ED)
    embt = jnp.transpose(emb)
    am, om = _channel_masks(b, hw)
    am = jnp.asarray(am)
    om = jnp.asarray(om)
    zsq3 = jnp.sum(zp ** 2, axis=1).reshape(ntok // _TOK_BLK, 1, _TOK_BLK)
    ridx2, loss_sum = _tc_vq(zp, embt, zsq3, am, om, 0, ntok // _TOK_BLK)
    loss = loss_sum[0, 0] * jnp.float32((1.0 + _BETA) / float(z.size))
    zq = _sc_gather(ridx2.reshape(ntok), emb)
    out = jnp.transpose(zq.reshape(hw, b, _ED), (1, 2, 0)).reshape(b, c, h, w)
    return loss, out
